# baseline (speedup 1.0000x reference)
import functools

import jax
import jax.numpy as jnp
from jax import lax
from jax.experimental import pallas as pl
from jax.experimental.pallas import tpu as pltpu
from jax.experimental.pallas import tpu_sc as plsc

EMBED_DIM = 32
NUM_EMBED = 8192
N_ROWS = 64 * 1024


_BM = 512


def _argmin_body(x_ref, e_ref, idx_ref, e2_ref):
    f = x_ref[...]

    @pl.when(pl.program_id(0) == 0)
    def _():
        e = e_ref[...]
        e2_ref[...] = jnp.sum(e * e, axis=0, keepdims=True)

    half = NUM_EMBED // 2
    fm2 = (-2.0 * f).astype(jnp.bfloat16)
    dn = (((1,), (0,)), ((), ()))
    simn2_0 = lax.dot_general(
        fm2, e_ref[:, :half].astype(jnp.bfloat16), dn,
        preferred_element_type=jnp.float32)
    simn2_1 = lax.dot_general(
        fm2, e_ref[:, half:].astype(jnp.bfloat16), dn,
        preferred_element_type=jnp.float32)
    f2 = jnp.sum(f * f, axis=1, keepdims=True)
    e2 = e2_ref[...]
    d0 = (f2 + e2[:, :half]) + simn2_0
    d1 = (f2 + e2[:, half:]) + simn2_1
    m0 = jnp.min(d0, axis=1, keepdims=True)
    m1 = jnp.min(d1, axis=1, keepdims=True)
    m0q = m0.astype(jnp.bfloat16).astype(jnp.float32)
    take1 = m1 < m0q
    nanv = jnp.float32(jnp.nan)
    t0 = jnp.where(take1, nanv, m0)
    t1 = jnp.where(take1, m1, nanv)
    ids0 = lax.broadcasted_iota(jnp.int32, (1, half), 1).astype(jnp.float32)
    ids1 = ids0 + jnp.float32(half)
    big = jnp.float32(NUM_EMBED)
    aw0 = jnp.min(jnp.where(d0 == t0, ids0, big), axis=1)
    aw1 = jnp.min(jnp.where(d1 == t1, ids1, big), axis=1)
    idx_ref[...] = jnp.minimum(aw0, aw1).astype(jnp.int32)


def _compute_indices(flat, embeddings):
    grid = N_ROWS // _BM
    return pl.pallas_call(
        _argmin_body,
        grid=(grid,),
        in_specs=[
            pl.BlockSpec((_BM, EMBED_DIM), lambda i: (i, 0)),
            pl.BlockSpec((EMBED_DIM, NUM_EMBED), lambda i: (0, 0)),
        ],
        out_specs=pl.BlockSpec((_BM,), lambda i: (i,)),
        out_shape=jax.ShapeDtypeStruct((N_ROWS,), jnp.int32),
        scratch_shapes=[pltpu.VMEM((1, NUM_EMBED), jnp.float32)],
    )(flat, embeddings)


_NC, _NS, _L = 2, 16, 16
_NW = _NC * _NS
_B_PER_W = N_ROWS // _NW
_CHUNK = 128
_NCHUNK = _B_PER_W // _CHUNK

_sc_mesh = plsc.VectorSubcoreMesh(core_axis_name="c", subcore_axis_name="s")


@functools.partial(
    pl.kernel,
    mesh=_sc_mesh,
    out_type=jax.ShapeDtypeStruct((N_ROWS, EMBED_DIM), jnp.float32),
    scratch_types=[
        pltpu.VMEM((_NCHUNK, _CHUNK), jnp.int32),
        pltpu.VMEM((_B_PER_W, EMBED_DIM), jnp.float32),
        pltpu.SemaphoreType.DMA,
    ],
    compiler_params=pltpu.CompilerParams(use_tc_tiling_on_sc=False),
)
def _sc_gather(table_hbm, idx_hbm, out_hbm, idx_v, rows_v, sem):
    wid = lax.axis_index("s") * _NC + lax.axis_index("c")
    base = wid * _B_PER_W
    pltpu.sync_copy(idx_hbm.at[pl.ds(wid * _NCHUNK, _NCHUNK), :], idx_v)
    copies = [
        pltpu.async_copy(
            table_hbm.at[idx_v.at[j]],
            rows_v.at[pl.ds(j * _CHUNK, _CHUNK), :],
            sem,
        )
        for j in range(_NCHUNK)
    ]
    for c in copies:
        c.wait()
    pltpu.sync_copy(rows_v, out_hbm.at[pl.ds(base, _B_PER_W)])


def kernel(x, embeddings):
    flat = x.reshape(-1, EMBED_DIM)
    idx = _compute_indices(flat, embeddings)
    table = embeddings.T
    quantized = _sc_gather(table, idx.reshape(N_ROWS // _CHUNK, _CHUNK))
    return quantized.reshape(x.shape)


# speedup vs baseline: 1.0470x; 1.0470x over previous
import functools

import jax
import jax.numpy as jnp
from jax import lax
from jax.experimental import pallas as pl
from jax.experimental.pallas import tpu as pltpu
from jax.experimental.pallas import tpu_sc as plsc

EMBED_DIM = 32
NUM_EMBED = 8192
N_ROWS = 64 * 1024


_BM = 512


def _argmin_body(x_ref, e_ref, idx_ref, e2_ref):
    f = x_ref[...]

    @pl.when(pl.program_id(0) == 0)
    def _():
        e = e_ref[...]
        e2_ref[...] = jnp.sum(e * e, axis=0, keepdims=True)

    simn2 = lax.dot_general(
        (-2.0 * f).astype(jnp.bfloat16), e_ref[...].astype(jnp.bfloat16),
        (((1,), (0,)), ((), ())),
        preferred_element_type=jnp.float32,
    )
    f2 = jnp.sum(f * f, axis=1, keepdims=True)
    d = (f2 + e2_ref[...]) + simn2
    half = NUM_EMBED // 2
    d0 = d[:, :half]
    d1 = d[:, half:]
    m0 = jnp.min(d0, axis=1, keepdims=True)
    m1 = jnp.min(d1, axis=1, keepdims=True)
    m0q = m0.astype(jnp.bfloat16).astype(jnp.float32)
    take1 = m1 < m0q
    dsel = jnp.where(take1, d1, d0)
    msel = jnp.where(take1, m1, m0)
    ids = lax.broadcasted_iota(jnp.int32, (1, half), 1).astype(jnp.float32)
    aw = jnp.min(jnp.where(dsel == msel, ids, jnp.float32(half)), axis=1)
    idx = aw.astype(jnp.int32) + jnp.where(take1[:, 0], half, 0)
    idx_ref[...] = idx


def _compute_indices(flat, embeddings):
    grid = N_ROWS // _BM
    return pl.pallas_call(
        _argmin_body,
        grid=(grid,),
        in_specs=[
            pl.BlockSpec((_BM, EMBED_DIM), lambda i: (i, 0)),
            pl.BlockSpec((EMBED_DIM, NUM_EMBED), lambda i: (0, 0)),
        ],
        out_specs=pl.BlockSpec((_BM,), lambda i: (i,)),
        out_shape=jax.ShapeDtypeStruct((N_ROWS,), jnp.int32),
        scratch_shapes=[pltpu.VMEM((1, NUM_EMBED), jnp.float32)],
    )(flat, embeddings)


_NC, _NS, _L = 2, 16, 16
_NW = _NC * _NS
_B_PER_W = N_ROWS // _NW
_CHUNK = 128
_NCHUNK = _B_PER_W // _CHUNK

_sc_mesh = plsc.VectorSubcoreMesh(core_axis_name="c", subcore_axis_name="s")


@functools.partial(
    pl.kernel,
    mesh=_sc_mesh,
    out_type=jax.ShapeDtypeStruct((N_ROWS, EMBED_DIM), jnp.float32),
    scratch_types=[
        pltpu.VMEM((_B_PER_W,), jnp.int32),
        pltpu.VMEM((_B_PER_W, EMBED_DIM), jnp.float32),
        pltpu.SemaphoreType.DMA,
    ],
    compiler_params=pltpu.CompilerParams(use_tc_tiling_on_sc=False),
)
def _sc_gather(table_hbm, idx_hbm, out_hbm, idx_v, rows_v, sem):
    wid = lax.axis_index("s") * _NC + lax.axis_index("c")
    base = wid * _B_PER_W
    pltpu.sync_copy(idx_hbm.at[pl.ds(base, _B_PER_W)], idx_v)
    copies = [
        pltpu.async_copy(
            table_hbm.at[idx_v.at[pl.ds(j * _CHUNK, _CHUNK)]],
            rows_v.at[pl.ds(j * _CHUNK, _CHUNK), :],
            sem,
        )
        for j in range(_NCHUNK)
    ]
    for c in copies:
        c.wait()
    pltpu.sync_copy(rows_v, out_hbm.at[pl.ds(base, _B_PER_W)])


def kernel(x, embeddings):
    flat = x.reshape(-1, EMBED_DIM)
    idx = _compute_indices(flat, embeddings)
    table = embeddings.T
    quantized = _sc_gather(table, idx)
    return quantized.reshape(x.shape)
